# Initial kernel scaffold; baseline (speedup 1.0000x reference)
import jax
import jax.numpy as jnp
from jax.experimental import pallas as pl


def kernel(x, W, b):
    raise NotImplementedError("write your pallas kernel here")


# speedup vs baseline: 2.0599x; 2.0599x over previous
import functools

import jax
import jax.numpy as jnp
from jax.experimental import pallas as pl

N_TOKENS = 32768
D_EMBED = 768
N_EXPERTS = 64
BLOCK = 1024


def _router_block(x_ref, wt_ref, b_ref, scores_ref, idx_ref):
    x_blk = x_ref[...]
    logits = jnp.dot(x_blk, wt_ref[...], preferred_element_type=jnp.float32)
    logits = logits + b_ref[...]

    lane = jax.lax.broadcasted_iota(jnp.int32, logits.shape, 1)
    m1 = jnp.max(logits, axis=1, keepdims=True)
    i1 = jnp.min(jnp.where(logits == m1, lane, N_EXPERTS), axis=1, keepdims=True)
    logits2 = jnp.where(lane == i1, -jnp.inf, logits)
    m2 = jnp.max(logits2, axis=1, keepdims=True)
    i2 = jnp.min(jnp.where(logits2 == m2, lane, N_EXPERTS), axis=1, keepdims=True)

    denom = jnp.sum(jnp.exp(logits - m1), axis=1, keepdims=True)
    s1 = 1.0 / denom
    s2 = jnp.exp(m2 - m1) / denom

    scores_ref[...] = jnp.concatenate([s1, s2], axis=1)
    idx_ref[...] = jnp.concatenate([i1, i2], axis=1)


@jax.jit
def kernel(x, W, b):
    wt = W.T
    b2 = b.reshape(1, N_EXPERTS)
    grid = (N_TOKENS // BLOCK,)
    scores, idx = pl.pallas_call(
        _router_block,
        grid=grid,
        in_specs=[
            pl.BlockSpec((BLOCK, D_EMBED), lambda i: (i, 0)),
            pl.BlockSpec((D_EMBED, N_EXPERTS), lambda i: (0, 0)),
            pl.BlockSpec((1, N_EXPERTS), lambda i: (0, 0)),
        ],
        out_specs=[
            pl.BlockSpec((BLOCK, 2), lambda i: (i, 0)),
            pl.BlockSpec((BLOCK, 2), lambda i: (i, 0)),
        ],
        out_shape=[
            jax.ShapeDtypeStruct((N_TOKENS, 2), jnp.float32),
            jax.ShapeDtypeStruct((N_TOKENS, 2), jnp.int32),
        ],
    )(x, wt, b2)
    return scores, idx


# speedup vs baseline: 2.3444x; 1.1381x over previous
import functools

import jax
import jax.numpy as jnp
from jax.experimental import pallas as pl

N_TOKENS = 32768
D_EMBED = 768
N_EXPERTS = 64
BLOCK = 2048


def _router_block(x_ref, wt_ref, b_ref, scores_ref, idx_ref):
    x_blk = x_ref[...]
    logits = jnp.dot(x_blk, wt_ref[...], preferred_element_type=jnp.float32)
    logits = logits + b_ref[...]

    lane = jax.lax.broadcasted_iota(jnp.int32, logits.shape, 1)
    m1 = jnp.max(logits, axis=1, keepdims=True)
    i1 = jnp.min(jnp.where(logits == m1, lane, N_EXPERTS), axis=1, keepdims=True)
    logits2 = jnp.where(lane == i1, -jnp.inf, logits)
    m2 = jnp.max(logits2, axis=1, keepdims=True)
    i2 = jnp.min(jnp.where(logits2 == m2, lane, N_EXPERTS), axis=1, keepdims=True)

    denom = jnp.sum(jnp.exp(logits - m1), axis=1, keepdims=True)
    s1 = 1.0 / denom
    s2 = jnp.exp(m2 - m1) / denom

    scores_ref[...] = jnp.concatenate([s1, s2], axis=1)
    idx_ref[...] = jnp.concatenate([i1, i2], axis=1)


@jax.jit
def kernel(x, W, b):
    wt = W.T
    b2 = b.reshape(1, N_EXPERTS)
    grid = (N_TOKENS // BLOCK,)
    scores, idx = pl.pallas_call(
        _router_block,
        grid=grid,
        in_specs=[
            pl.BlockSpec((BLOCK, D_EMBED), lambda i: (i, 0)),
            pl.BlockSpec((D_EMBED, N_EXPERTS), lambda i: (0, 0)),
            pl.BlockSpec((1, N_EXPERTS), lambda i: (0, 0)),
        ],
        out_specs=[
            pl.BlockSpec((BLOCK, 2), lambda i: (i, 0)),
            pl.BlockSpec((BLOCK, 2), lambda i: (i, 0)),
        ],
        out_shape=[
            jax.ShapeDtypeStruct((N_TOKENS, 2), jnp.float32),
            jax.ShapeDtypeStruct((N_TOKENS, 2), jnp.int32),
        ],
    )(x, wt, b2)
    return scores, idx


# speedup vs baseline: 2.5535x; 1.0892x over previous
import functools

import jax
import jax.numpy as jnp
from jax.experimental import pallas as pl

N_TOKENS = 32768
D_EMBED = 768
N_EXPERTS = 64
BLOCK = 4096


def _router_block(x_ref, wt_ref, b_ref, scores_ref, idx_ref):
    x_blk = x_ref[...]
    logits = jnp.dot(x_blk, wt_ref[...], preferred_element_type=jnp.float32)
    logits = logits + b_ref[...]

    lane = jax.lax.broadcasted_iota(jnp.int32, logits.shape, 1)
    m1 = jnp.max(logits, axis=1, keepdims=True)
    i1 = jnp.min(jnp.where(logits == m1, lane, N_EXPERTS), axis=1, keepdims=True)
    logits2 = jnp.where(lane == i1, -jnp.inf, logits)
    m2 = jnp.max(logits2, axis=1, keepdims=True)
    i2 = jnp.min(jnp.where(logits2 == m2, lane, N_EXPERTS), axis=1, keepdims=True)

    denom = jnp.sum(jnp.exp(logits - m1), axis=1, keepdims=True)
    s1 = 1.0 / denom
    s2 = jnp.exp(m2 - m1) / denom

    scores_ref[...] = jnp.concatenate([s1, s2], axis=1)
    idx_ref[...] = jnp.concatenate([i1, i2], axis=1)


@jax.jit
def kernel(x, W, b):
    wt = W.T
    b2 = b.reshape(1, N_EXPERTS)
    grid = (N_TOKENS // BLOCK,)
    scores, idx = pl.pallas_call(
        _router_block,
        grid=grid,
        in_specs=[
            pl.BlockSpec((BLOCK, D_EMBED), lambda i: (i, 0)),
            pl.BlockSpec((D_EMBED, N_EXPERTS), lambda i: (0, 0)),
            pl.BlockSpec((1, N_EXPERTS), lambda i: (0, 0)),
        ],
        out_specs=[
            pl.BlockSpec((BLOCK, 2), lambda i: (i, 0)),
            pl.BlockSpec((BLOCK, 2), lambda i: (i, 0)),
        ],
        out_shape=[
            jax.ShapeDtypeStruct((N_TOKENS, 2), jnp.float32),
            jax.ShapeDtypeStruct((N_TOKENS, 2), jnp.int32),
        ],
    )(x, wt, b2)
    return scores, idx


# speedup vs baseline: 2.6667x; 1.0443x over previous
import functools

import jax
import jax.numpy as jnp
from jax.experimental import pallas as pl

N_TOKENS = 32768
D_EMBED = 768
N_EXPERTS = 64
BLOCK = 4096


def _router_block(x_ref, wt_ref, b_ref, scores_ref, idx_ref):
    x_blk = x_ref[...]
    logits = jnp.dot(x_blk, wt_ref[...], preferred_element_type=jnp.float32)
    logits = logits + b_ref[...]

    lane_f = jax.lax.broadcasted_iota(jnp.int32, logits.shape, 1).astype(jnp.float32)
    m1 = jnp.max(logits, axis=1, keepdims=True)
    i1f = jnp.min(jnp.where(logits == m1, lane_f, 64.0), axis=1, keepdims=True)
    logits2 = jnp.where(lane_f == i1f, -jnp.inf, logits)
    m2 = jnp.max(logits2, axis=1, keepdims=True)
    i2f = jnp.min(jnp.where(logits2 == m2, lane_f, 64.0), axis=1, keepdims=True)

    denom = jnp.sum(jnp.exp(logits - m1), axis=1, keepdims=True)
    s1 = 1.0 / denom
    s2 = jnp.exp(m2 - m1) / denom

    scores_ref[...] = jnp.concatenate([s1, s2], axis=1)
    idx_ref[...] = jnp.concatenate([i1f, i2f], axis=1).astype(jnp.int32)


@jax.jit
def kernel(x, W, b):
    wt = W.T
    b2 = b.reshape(1, N_EXPERTS)
    grid = (N_TOKENS // BLOCK,)
    scores, idx = pl.pallas_call(
        _router_block,
        grid=grid,
        in_specs=[
            pl.BlockSpec((BLOCK, D_EMBED), lambda i: (i, 0)),
            pl.BlockSpec((D_EMBED, N_EXPERTS), lambda i: (0, 0)),
            pl.BlockSpec((1, N_EXPERTS), lambda i: (0, 0)),
        ],
        out_specs=[
            pl.BlockSpec((BLOCK, 2), lambda i: (i, 0)),
            pl.BlockSpec((BLOCK, 2), lambda i: (i, 0)),
        ],
        out_shape=[
            jax.ShapeDtypeStruct((N_TOKENS, 2), jnp.float32),
            jax.ShapeDtypeStruct((N_TOKENS, 2), jnp.int32),
        ],
    )(x, wt, b2)
    return scores, idx
